# baseline (speedup 1.0000x reference)
import functools
import jax
import jax.numpy as jnp
from jax import lax
from jax.experimental import pallas as pl
from jax.experimental.pallas import tpu as pltpu
from jax.experimental.pallas import tpu_sc as plsc

D = 64
SCALE = 8.0
NC, NS = 2, 16
NW = NC * NS
SEQ = 200
BATCH = 4096
B = BATCH * SEQ
V = 1000000
BPW = B // NW
NCHW = BPW // 128
TD, DR = D // 8, 8
NTB = BATCH // 128
NVC = 7813
KA = 246

_mesh = plsc.VectorSubcoreMesh(
    core_axis_name="c", subcore_axis_name="s", num_cores=NC, num_subcores=NS
)
_params = pltpu.CompilerParams(needs_layout_passes=False)


@functools.partial(
    pl.kernel,
    out_type=jax.ShapeDtypeStruct((V // 2, 128), jnp.float32),
    mesh=_mesh,
    scratch_types=[
        [pltpu.VMEM((D, 128), jnp.float32) for _ in range(2)],
        [pltpu.VMEM((D, 128), jnp.float32) for _ in range(2)],
        [pltpu.SemaphoreType.DMA for _ in range(2)],
        [pltpu.SemaphoreType.DMA for _ in range(2)],
    ],
    compiler_params=_params,
)
def _convert(lutt_hbm, out_hbm, tin, tout, isem, osem):
    wid = lax.axis_index("s") * NC + lax.axis_index("c")
    iota = lax.iota(jnp.int32, 16)

    def fire_in(c, r):
        vc = pl.multiple_of(c * 128, 128)
        for d8 in range(0, D, 8):
            pltpu.async_copy(
                lutt_hbm.at[pl.ds(d8, 8), pl.ds(vc, 128)],
                tin[r].at[pl.ds(d8, 8)],
                isem[r],
            )

    def drain_in(c, r):
        vc = pl.multiple_of(c * 128, 128)
        for d8 in range(0, D, 8):
            pltpu.make_async_copy(
                lutt_hbm.at[pl.ds(d8, 8), pl.ds(vc, 128)],
                tin[r].at[pl.ds(d8, 8)],
                isem[r],
            ).wait()

    def transform(r):
        @pl.loop(0, D // 2)
        def _rows(pp):
            for psub in range(2):
                for h in range(2):
                    col = jnp.full((16,), h, jnp.int32) + (pp * 4 + psub * 2)
                    for cs in range(4):
                        v = plsc.load_gather(tin[r], [iota + cs * 16, col])
                        tout[r][
                            pp * 2 + psub, pl.ds(h * 64 + cs * 16, 16)
                        ] = v * SCALE

    def _out_copy(c, r, rows_n):
        return pltpu.make_async_copy(
            tout[r].at[pl.ds(0, rows_n)],
            out_hbm.at[pl.ds(pl.multiple_of(c * 64, 8), rows_n)],
            osem[r],
        )

    def fire_out(c, r):
        @pl.when(c < NVC - 1)
        def _():
            _out_copy(c, r, D).start()

        @pl.when(c == NVC - 1)
        def _():
            _out_copy(c, r, D // 2).start()

    def drain_out(c, r):
        @pl.when(c < NVC - 1)
        def _():
            _out_copy(c, r, D).wait()

        @pl.when(c == NVC - 1)
        def _():
            _out_copy(c, r, D // 2).wait()

    fire_in(wid, 0)
    fire_in(wid + 32, 1)

    @pl.loop(0, KA // 2)
    def _step(j):
        for r in range(2):
            k = 2 * j + r
            c = wid + 32 * k

            @pl.when(c < NVC)
            def _():
                drain_in(c, r)

            @pl.when(jnp.logical_and(k >= 2, c - 64 < NVC))
            def _():
                drain_out(c - 64, r)

            @pl.when(c < NVC)
            def _():
                transform(r)

            @pl.when(c + 64 < NVC)
            def _():
                fire_in(c + 64, r)

            @pl.when(c < NVC)
            def _():
                fire_out(c, r)

    for k in (KA - 2, KA - 1):
        ce = wid + 32 * k

        @pl.when(ce < NVC)
        def _():
            drain_out(ce, k % 2)


@functools.partial(
    pl.kernel,
    out_type=jax.ShapeDtypeStruct((SEQ * D * BATCH,), jnp.float32),
    mesh=_mesh,
    scratch_types=[
        pltpu.VMEM((BPW,), jnp.int32),
        pltpu.VMEM((BPW,), jnp.int32),
        [pltpu.VMEM((128, 128), jnp.float32) for _ in range(2)],
        [pltpu.VMEM((TD * DR * 128,), jnp.float32) for _ in range(2)],
        [pltpu.SemaphoreType.DMA for _ in range(2)],
        [pltpu.SemaphoreType.DMA for _ in range(2)],
    ],
    compiler_params=_params,
)
def _gather(x_hbm, lutp_hbm, out_hbm, idx_v, line_v, rows, stage, gsem, osem):
    wid = lax.axis_index("s") * NC + lax.axis_index("c")
    iota = lax.iota(jnp.int32, 16)

    pltpu.sync_copy(x_hbm.at[wid], idx_v)

    @pl.loop(0, BPW // 16)
    def _halve(i):
        line_v[pl.ds(i * 16, 16)] = idx_v[pl.ds(i * 16, 16)] >> 1

    def fire_gather(k, r):
        pltpu.async_copy(
            lutp_hbm.at[line_v.at[pl.ds(k * 128, 128)]], rows[r], gsem[r]
        )

    def drain_gather(k, r):
        pltpu.make_async_copy(
            lutp_hbm.at[line_v.at[pl.ds(k * 128, 128)]], rows[r], gsem[r]
        ).wait()

    def transform(k, r):
        @pl.loop(0, 8)
        def _bblock(bb):
            b0 = bb * 16
            halves = (idx_v[pl.ds(k * 128 + b0, 16)] & 1) << 6
            row_ids = iota + b0
            for og in range(8):
                colg = halves + og * 8
                for oo in range(8):
                    o = og * 8 + oo
                    v = plsc.load_gather(rows[r], [row_ids, colg + oo])
                    doff = (o >> 3) * 1024 + (o & 7) * 128
                    stage[r][pl.ds(doff + b0, 16)] = v

    def _write_copies(c, r):
        s = c >> 5
        tb = c & 31
        base = s * (D * BATCH) + tb * 1024
        return [
            pltpu.make_async_copy(
                stage[r].at[pl.ds(td * 1024, 1024)],
                out_hbm.at[
                    pl.ds(pl.multiple_of(base + td * (DR * BATCH), 1024), 1024)
                ],
                osem[r],
            )
            for td in range(TD)
        ]

    def fire_write(c, r):
        for cp in _write_copies(c, r):
            cp.start()

    def drain_write(c, r):
        for cp in _write_copies(c, r):
            cp.wait()

    c0 = wid * NCHW
    fire_gather(0, 0)
    fire_gather(1, 1)

    @pl.loop(0, NCHW // 2)
    def _step(j):
        for r in range(2):
            k = 2 * j + r
            drain_gather(k, r)

            @pl.when(k >= 2)
            def _():
                drain_write(c0 + k - 2, r)

            transform(k, r)

            @pl.when(k + 2 < NCHW)
            def _():
                fire_gather(k + 2, r)

            fire_write(c0 + k, r)

    drain_write(c0 + NCHW - 2, 0)
    drain_write(c0 + NCHW - 1, 1)


def kernel(x, lut):
    lutp = _convert(lut.T)
    xf = x.T.reshape(NW, BPW).astype(jnp.int32)
    flat = _gather(xf, lutp)
    out5 = flat.reshape(SEQ, TD, NTB, DR, 128)
    return out5.transpose(2, 4, 0, 1, 3).reshape(BATCH, SEQ, D)


# speedup vs baseline: 1.6042x; 1.6042x over previous
import functools
import jax
import jax.numpy as jnp
from jax import lax
from jax.experimental import pallas as pl
from jax.experimental.pallas import tpu as pltpu
from jax.experimental.pallas import tpu_sc as plsc

D = 64
SCALE = 8.0
NC, NS = 2, 16
NW = NC * NS
SEQ = 200
BATCH = 4096
B = BATCH * SEQ
V = 1000000
BPW = B // NW
NCHW = BPW // 128
TD, DR = D // 8, 8
NTB = BATCH // 128
NVC = 7813
KA = 246

_mesh = plsc.VectorSubcoreMesh(
    core_axis_name="c", subcore_axis_name="s", num_cores=NC, num_subcores=NS
)
_params = pltpu.CompilerParams(needs_layout_passes=False)


@functools.partial(
    pl.kernel,
    out_type=jax.ShapeDtypeStruct((V // 2, 128), jnp.float32),
    mesh=_mesh,
    scratch_types=[
        [pltpu.VMEM((D, 128), jnp.float32) for _ in range(2)],
        [pltpu.VMEM((D, 128), jnp.float32) for _ in range(2)],
        [pltpu.SemaphoreType.DMA for _ in range(2)],
        [pltpu.SemaphoreType.DMA for _ in range(2)],
    ],
    compiler_params=_params,
)
def _convert(lutt_hbm, out_hbm, tin, tout, isem, osem):
    wid = lax.axis_index("s") * NC + lax.axis_index("c")
    iota = lax.iota(jnp.int32, 16)

    def fire_in(c, r):
        vc = pl.multiple_of(c * 128, 128)
        for d8 in range(0, D, 8):
            pltpu.async_copy(
                lutt_hbm.at[pl.ds(d8, 8), pl.ds(vc, 128)],
                tin[r].at[pl.ds(d8, 8)],
                isem[r],
            )

    def drain_in(c, r):
        vc = pl.multiple_of(c * 128, 128)
        for d8 in range(0, D, 8):
            pltpu.make_async_copy(
                lutt_hbm.at[pl.ds(d8, 8), pl.ds(vc, 128)],
                tin[r].at[pl.ds(d8, 8)],
                isem[r],
            ).wait()

    def transform(r):
        @pl.loop(0, D // 2)
        def _rows(pp):
            vs = []
            for psub in range(2):
                for h in range(2):
                    col = jnp.full((16,), h, jnp.int32) + (pp * 4 + psub * 2)
                    for cs in range(4):
                        v = plsc.load_gather(tin[r], [iota + cs * 16, col])
                        vs.append((psub, h, cs, v * SCALE))
            for psub, h, cs, v in vs:
                tout[r][pp * 2 + psub, pl.ds(h * 64 + cs * 16, 16)] = v

    def _out_copy(c, r, rows_n):
        return pltpu.make_async_copy(
            tout[r].at[pl.ds(0, rows_n)],
            out_hbm.at[pl.ds(pl.multiple_of(c * 64, 8), rows_n)],
            osem[r],
        )

    def fire_out(c, r):
        @pl.when(c < NVC - 1)
        def _():
            _out_copy(c, r, D).start()

        @pl.when(c == NVC - 1)
        def _():
            _out_copy(c, r, D // 2).start()

    def drain_out(c, r):
        @pl.when(c < NVC - 1)
        def _():
            _out_copy(c, r, D).wait()

        @pl.when(c == NVC - 1)
        def _():
            _out_copy(c, r, D // 2).wait()

    fire_in(wid, 0)
    fire_in(wid + 32, 1)

    @pl.loop(0, KA // 2)
    def _step(j):
        for r in range(2):
            k = 2 * j + r
            c = wid + 32 * k

            @pl.when(c < NVC)
            def _():
                drain_in(c, r)

            @pl.when(jnp.logical_and(k >= 2, c - 64 < NVC))
            def _():
                drain_out(c - 64, r)

            @pl.when(c < NVC)
            def _():
                transform(r)

            @pl.when(c + 64 < NVC)
            def _():
                fire_in(c + 64, r)

            @pl.when(c < NVC)
            def _():
                fire_out(c, r)

    for k in (KA - 2, KA - 1):
        ce = wid + 32 * k

        @pl.when(ce < NVC)
        def _():
            drain_out(ce, k % 2)


@functools.partial(
    pl.kernel,
    out_type=jax.ShapeDtypeStruct((SEQ * D * BATCH,), jnp.float32),
    mesh=_mesh,
    scratch_types=[
        pltpu.VMEM((BPW,), jnp.int32),
        pltpu.VMEM((BPW,), jnp.int32),
        [pltpu.VMEM((128, 128), jnp.float32) for _ in range(2)],
        [pltpu.VMEM((TD * DR * 128,), jnp.float32) for _ in range(2)],
        [pltpu.SemaphoreType.DMA for _ in range(2)],
        [pltpu.SemaphoreType.DMA for _ in range(2)],
    ],
    compiler_params=_params,
)
def _gather(x_hbm, lutp_hbm, out_hbm, idx_v, line_v, rows, stage, gsem, osem):
    wid = lax.axis_index("s") * NC + lax.axis_index("c")
    iota = lax.iota(jnp.int32, 16)

    pltpu.sync_copy(x_hbm.at[wid], idx_v)

    @pl.loop(0, BPW // 16)
    def _halve(i):
        line_v[pl.ds(i * 16, 16)] = idx_v[pl.ds(i * 16, 16)] >> 1

    def fire_gather(k, r):
        pltpu.async_copy(
            lutp_hbm.at[line_v.at[pl.ds(k * 128, 128)]], rows[r], gsem[r]
        )

    def drain_gather(k, r):
        pltpu.make_async_copy(
            lutp_hbm.at[line_v.at[pl.ds(k * 128, 128)]], rows[r], gsem[r]
        ).wait()

    def transform(k, r):
        @pl.loop(0, 8)
        def _bblock(bb):
            b0 = bb * 16
            halves = (idx_v[pl.ds(k * 128 + b0, 16)] & 1) << 6
            row_ids = iota + b0
            for og in range(8):
                colg = halves + og * 8
                vs = []
                for oo in range(8):
                    o = og * 8 + oo
                    v = plsc.load_gather(rows[r], [row_ids, colg + oo])
                    vs.append((o, v))
                for o, v in vs:
                    doff = (o >> 3) * 1024 + (o & 7) * 128
                    stage[r][pl.ds(doff + b0, 16)] = v

    def _write_copies(c, r):
        s = c >> 5
        tb = c & 31
        base = s * (D * BATCH) + tb * 1024
        return [
            pltpu.make_async_copy(
                stage[r].at[pl.ds(td * 1024, 1024)],
                out_hbm.at[
                    pl.ds(pl.multiple_of(base + td * (DR * BATCH), 1024), 1024)
                ],
                osem[r],
            )
            for td in range(TD)
        ]

    def fire_write(c, r):
        for cp in _write_copies(c, r):
            cp.start()

    def drain_write(c, r):
        for cp in _write_copies(c, r):
            cp.wait()

    c0 = wid * NCHW
    fire_gather(0, 0)
    fire_gather(1, 1)

    @pl.loop(0, NCHW // 2)
    def _step(j):
        for r in range(2):
            k = 2 * j + r
            drain_gather(k, r)

            @pl.when(k >= 2)
            def _():
                drain_write(c0 + k - 2, r)

            transform(k, r)

            @pl.when(k + 2 < NCHW)
            def _():
                fire_gather(k + 2, r)

            fire_write(c0 + k, r)

    drain_write(c0 + NCHW - 2, 0)
    drain_write(c0 + NCHW - 1, 1)


def kernel(x, lut):
    lutp = _convert(lut.T)
    xf = x.T.reshape(NW, BPW).astype(jnp.int32)
    flat = _gather(xf, lutp)
    out5 = flat.reshape(SEQ, TD, NTB, DR, 128)
    return out5.transpose(2, 4, 0, 1, 3).reshape(BATCH, SEQ, D)


# speedup vs baseline: 1.6061x; 1.0012x over previous
import functools
import jax
import jax.numpy as jnp
from jax import lax
from jax.experimental import pallas as pl
from jax.experimental.pallas import tpu as pltpu
from jax.experimental.pallas import tpu_sc as plsc

D = 64
SCALE = 8.0
NC, NS = 2, 16
NW = NC * NS
SEQ = 200
BATCH = 4096
B = BATCH * SEQ
V = 1000000
BPW = B // NW
NCHW = BPW // 128
TD, DR = D // 8, 8
NTB = BATCH // 128
NVC = 7813
KA = 246

_mesh = plsc.VectorSubcoreMesh(
    core_axis_name="c", subcore_axis_name="s", num_cores=NC, num_subcores=NS
)
_params = pltpu.CompilerParams(needs_layout_passes=False)


@functools.partial(
    pl.kernel,
    out_type=jax.ShapeDtypeStruct((V // 2, 128), jnp.float32),
    mesh=_mesh,
    scratch_types=[
        [pltpu.VMEM((D, 128), jnp.float32) for _ in range(2)],
        [pltpu.VMEM((D, 128), jnp.float32) for _ in range(2)],
        [pltpu.SemaphoreType.DMA for _ in range(2)],
        [pltpu.SemaphoreType.DMA for _ in range(2)],
    ],
    compiler_params=_params,
)
def _convert(lutt_hbm, out_hbm, tin, tout, isem, osem):
    wid = lax.axis_index("s") * NC + lax.axis_index("c")
    iota = lax.iota(jnp.int32, 16)

    def fire_in(c, r):
        vc = pl.multiple_of(c * 128, 128)
        pltpu.async_copy(
            lutt_hbm.at[:, pl.ds(vc, 128)], tin[r], isem[r]
        )

    def drain_in(c, r):
        vc = pl.multiple_of(c * 128, 128)
        pltpu.make_async_copy(
            lutt_hbm.at[:, pl.ds(vc, 128)], tin[r], isem[r]
        ).wait()

    def transform(r):
        @pl.loop(0, D // 2)
        def _rows(pp):
            vs = []
            for psub in range(2):
                for h in range(2):
                    col = jnp.full((16,), h, jnp.int32) + (pp * 4 + psub * 2)
                    for cs in range(4):
                        v = plsc.load_gather(tin[r], [iota + cs * 16, col])
                        vs.append((psub, h, cs, v * SCALE))
            for psub, h, cs, v in vs:
                tout[r][pp * 2 + psub, pl.ds(h * 64 + cs * 16, 16)] = v

    def _out_copy(c, r, rows_n):
        return pltpu.make_async_copy(
            tout[r].at[pl.ds(0, rows_n)],
            out_hbm.at[pl.ds(pl.multiple_of(c * 64, 8), rows_n)],
            osem[r],
        )

    def fire_out(c, r):
        @pl.when(c < NVC - 1)
        def _():
            _out_copy(c, r, D).start()

        @pl.when(c == NVC - 1)
        def _():
            _out_copy(c, r, D // 2).start()

    def drain_out(c, r):
        @pl.when(c < NVC - 1)
        def _():
            _out_copy(c, r, D).wait()

        @pl.when(c == NVC - 1)
        def _():
            _out_copy(c, r, D // 2).wait()

    fire_in(wid, 0)
    fire_in(wid + 32, 1)

    @pl.loop(0, KA // 2)
    def _step(j):
        for r in range(2):
            k = 2 * j + r
            c = wid + 32 * k

            @pl.when(c < NVC)
            def _():
                drain_in(c, r)

            @pl.when(jnp.logical_and(k >= 2, c - 64 < NVC))
            def _():
                drain_out(c - 64, r)

            @pl.when(c < NVC)
            def _():
                transform(r)

            @pl.when(c + 64 < NVC)
            def _():
                fire_in(c + 64, r)

            @pl.when(c < NVC)
            def _():
                fire_out(c, r)

    for k in (KA - 2, KA - 1):
        ce = wid + 32 * k

        @pl.when(ce < NVC)
        def _():
            drain_out(ce, k % 2)


CH2 = 256
NCH2 = BPW // CH2


@functools.partial(
    pl.kernel,
    out_type=jax.ShapeDtypeStruct((SEQ * D * BATCH,), jnp.float32),
    mesh=_mesh,
    scratch_types=[
        pltpu.VMEM((BPW,), jnp.int32),
        [pltpu.VMEM((CH2,), jnp.int32) for _ in range(2)],
        [pltpu.VMEM((CH2, 128), jnp.float32) for _ in range(2)],
        [pltpu.VMEM((CH2 * D,), jnp.float32) for _ in range(2)],
        [pltpu.SemaphoreType.DMA for _ in range(2)],
        [pltpu.SemaphoreType.DMA for _ in range(2)],
    ],
    compiler_params=_params,
)
def _gather(x_hbm, lutp_hbm, out_hbm, idx_v, line, rows, stage, gsem, osem):
    wid = lax.axis_index("s") * NC + lax.axis_index("c")
    iota = lax.iota(jnp.int32, 16)

    pltpu.sync_copy(x_hbm.at[wid], idx_v)

    def compute_lines(k, r):
        for i in range(CH2 // 16):
            line[r][pl.ds(i * 16, 16)] = (
                idx_v[pl.ds(k * CH2 + i * 16, 16)] >> 1
            )

    def fire_gather(r):
        for g in range(CH2 // 128):
            pltpu.async_copy(
                lutp_hbm.at[line[r].at[pl.ds(g * 128, 128)]],
                rows[r].at[pl.ds(g * 128, 128)],
                gsem[r],
            )

    def drain_gather(r):
        for g in range(CH2 // 128):
            pltpu.make_async_copy(
                lutp_hbm.at[line[r].at[pl.ds(g * 128, 128)]],
                rows[r].at[pl.ds(g * 128, 128)],
                gsem[r],
            ).wait()

    def transform(k, r):
        @pl.loop(0, CH2 // 16)
        def _bblock(bb):
            b0 = bb * 16
            halves = (idx_v[pl.ds(k * CH2 + b0, 16)] & 1) << 6
            row_ids = iota + b0
            sbase = (bb >> 3) * 1024 + (bb & 7) * 16
            for og in range(8):
                colg = halves + og * 8
                vs = []
                for oo in range(8):
                    o = og * 8 + oo
                    v = plsc.load_gather(rows[r], [row_ids, colg + oo])
                    vs.append((o, v))
                for o, v in vs:
                    doff = (o >> 3) * 2048 + (o & 7) * 128
                    stage[r][pl.ds(sbase + doff, 16)] = v

    def _write_copies(c, r):
        s = c >> 4
        tb0 = (c & 15) * 2
        base = s * (D * BATCH) + tb0 * 1024
        return [
            pltpu.make_async_copy(
                stage[r].at[pl.ds(td * 2048, 2048)],
                out_hbm.at[
                    pl.ds(pl.multiple_of(base + td * (DR * BATCH), 1024), 2048)
                ],
                osem[r],
            )
            for td in range(TD)
        ]

    def fire_write(c, r):
        for cp in _write_copies(c, r):
            cp.start()

    def drain_write(c, r):
        for cp in _write_copies(c, r):
            cp.wait()

    c0 = wid * NCH2
    compute_lines(0, 0)
    fire_gather(0)
    compute_lines(1, 1)
    fire_gather(1)

    @pl.loop(0, NCH2 // 2)
    def _step(j):
        for r in range(2):
            k = 2 * j + r
            drain_gather(r)

            @pl.when(k >= 2)
            def _():
                drain_write(c0 + k - 2, r)

            transform(k, r)

            @pl.when(k + 2 < NCH2)
            def _():
                compute_lines(k + 2, r)
                fire_gather(r)

            fire_write(c0 + k, r)

    drain_write(c0 + NCH2 - 2, 0)
    drain_write(c0 + NCH2 - 1, 1)


def kernel(x, lut):
    lutp = _convert(lut.T)
    xf = x.T.reshape(NW, BPW).astype(jnp.int32)
    flat = _gather(xf, lutp)
    out5 = flat.reshape(SEQ, TD, NTB, DR, 128)
    return out5.transpose(2, 4, 0, 1, 3).reshape(BATCH, SEQ, D)
